# baseline (speedup 1.0000x reference)
import math

import jax
import jax.numpy as jnp
from jax.experimental import pallas as pl

_SEG = 4


def _fused_body(x_ref, ctx_ref, W0_ref, b0_ref, W1_ref, b1_ref,
                Wkv_ref, bkv_ref, Wq_ref, bq_ref,
                emb_ref, w_ref):
    H = W1_ref.shape[1]
    S, L, D = x_ref.shape
    x = x_ref[...].reshape(S * L, D)
    h = jnp.maximum(jnp.dot(x, W0_ref[...], preferred_element_type=jnp.float32)
                    + b0_ref[...], 0.0)
    h = jnp.maximum(jnp.dot(h, W1_ref[...], preferred_element_type=jnp.float32)
                    + b1_ref[...], 0.0)
    kv = jnp.dot(h, Wkv_ref[...], preferred_element_type=jnp.float32) + bkv_ref[...]
    key = kv[:, :H]
    value = kv[:, H:]
    q = jnp.dot(ctx_ref[:, 0, :], Wq_ref[...],
                preferred_element_type=jnp.float32) + bq_ref[...]
    logits_all = jnp.dot(key, q.T,
                         preferred_element_type=jnp.float32)
    eye = jnp.eye(S, dtype=jnp.float32)
    logits = jnp.sum(logits_all.reshape(S, L, S) * eye[:, None, :],
                     axis=-1) * (1.0 / math.sqrt(H))
    m = jnp.max(logits, axis=1, keepdims=True)
    ex = jnp.exp(logits - m)
    s = jnp.sum(ex, axis=1, keepdims=True)
    w = ex / s
    for i in range(S):
        emb_ref[i] = jnp.dot(w[i:i + 1, :], value[i * L:(i + 1) * L, :],
                             preferred_element_type=jnp.float32)
    w_ref[...] = w[:, None, :]


def kernel(objects_list, context, W0, b0, W1, b1, Wkv, bkv, Wq, bq):
    B, L, D = objects_list.shape
    D_CTX = context.shape[1]
    H = W1.shape[1]
    S = _SEG
    ctx3 = context.reshape(B, 1, D_CTX)
    b0r = b0.reshape(1, -1)
    b1r = b1.reshape(1, -1)
    bkvr = bkv.reshape(1, -1)
    bqr = bq.reshape(1, -1)

    full = lambda shape: pl.BlockSpec(shape, lambda b: (0,) * len(shape))
    emb, w = pl.pallas_call(
        _fused_body,
        grid=(B // S,),
        in_specs=[
            pl.BlockSpec((S, L, D), lambda b: (b, 0, 0)),
            pl.BlockSpec((S, 1, D_CTX), lambda b: (b, 0, 0)),
            full(W0.shape), full(b0r.shape),
            full(W1.shape), full(b1r.shape),
            full(Wkv.shape), full(bkvr.shape),
            full(Wq.shape), full(bqr.shape),
        ],
        out_specs=[
            pl.BlockSpec((S, 1, H), lambda b: (b, 0, 0)),
            pl.BlockSpec((S, 1, L), lambda b: (b, 0, 0)),
        ],
        out_shape=[
            jax.ShapeDtypeStruct((B, 1, H), jnp.float32),
            jax.ShapeDtypeStruct((B, 1, L), jnp.float32),
        ],
    )(objects_list, ctx3, W0, b0r, W1, b1r, Wkv, bkvr, Wq, bqr)
    return emb.reshape(B, H), w.reshape(B, L)
